# baseline (speedup 1.0000x reference)
import functools

import jax
import jax.numpy as jnp
from jax import lax
from jax.experimental import pallas as pl
from jax.experimental.pallas import tpu as pltpu
from jax.experimental.pallas import tpu_sc as plsc

_NC = 2
_NS = 16
_NW = _NC * _NS

_C = 128
_NBUF = 4


@functools.lru_cache(maxsize=None)
def _make_gather(n, v, d):
    per_w = n // _NW
    n_chunks = per_w // _C
    n_groups = n_chunks // _NBUF
    assert per_w * _NW == n and n_chunks * _C == per_w and n_groups * _NBUF == n_chunks

    mesh = plsc.VectorSubcoreMesh(
        core_axis_name="c", subcore_axis_name="s", num_cores=_NC, num_subcores=_NS
    )

    @functools.partial(
        pl.kernel,
        out_type=jax.ShapeDtypeStruct((n, d), jnp.float32),
        mesh=mesh,
        compiler_params=pltpu.CompilerParams(use_tc_tiling_on_sc=False),
        scratch_types=[
            pltpu.VMEM((n_chunks, _C), jnp.int32),
            pltpu.VMEM((_NBUF, _C, d), jnp.float32),
            [pltpu.SemaphoreType.DMA] * _NBUF,
            [pltpu.SemaphoreType.DMA] * _NBUF,
        ],
    )
    def gather_kernel(table_hbm, idx_hbm, out_hbm, idx_v, rows_v, gsems, wsems):
        wid = lax.axis_index("s") * _NC + lax.axis_index("c")
        base = wid * per_w
        pltpu.sync_copy(idx_hbm.at[wid], idx_v)

        def issue_gather(g, b):
            pltpu.async_copy(table_hbm.at[idx_v.at[g]], rows_v.at[b], gsems[b])

        def wait_gather(b):
            pltpu.make_async_copy(
                table_hbm.at[idx_v.at[0]], rows_v.at[b], gsems[b]
            ).wait()

        def issue_write(g, b):
            pltpu.async_copy(
                rows_v.at[b], out_hbm.at[pl.ds(base + g * _C, _C)], wsems[b]
            )

        def wait_write(b):
            pltpu.make_async_copy(
                rows_v.at[b], out_hbm.at[pl.ds(base, _C)], wsems[b]
            ).wait()

        for b in range(_NBUF):
            issue_gather(b, b)

        def group(gi, carry):
            for b in range(_NBUF):
                wait_gather(b)
                issue_write(gi * _NBUF + b, b)

            @pl.when(gi + 1 < n_groups)
            def _next():
                for b in range(_NBUF):
                    wait_write(b)
                    issue_gather((gi + 1) * _NBUF + b, b)

            return carry

        lax.fori_loop(0, n_groups, group, 0)
        for b in range(_NBUF):
            wait_write(b)

    return gather_kernel


def kernel(x, table):
    b, l = x.shape
    v, d = table.shape
    n = b * l
    per_w = n // _NW
    idx = x.reshape(_NW, per_w // _C, _C)
    out = _make_gather(n, v, d)(table, idx)
    return out.reshape(b, l, d)


# speedup vs baseline: 1.0015x; 1.0015x over previous
import functools

import jax
import jax.numpy as jnp
from jax import lax
from jax.experimental import pallas as pl
from jax.experimental.pallas import tpu as pltpu
from jax.experimental.pallas import tpu_sc as plsc

_NC = 2
_NS = 16
_NW = _NC * _NS

_C = 256
_K = 1
_M = _C * _K
_NSET = 4
_LAG = _NSET - 1


@functools.lru_cache(maxsize=None)
def _make_gather(n, v, d):
    per_w = n // _NW
    n_chunks = per_w // _C
    n_sg = per_w // _M
    assert per_w * _NW == n and n_chunks * _C == per_w and n_sg * _M == per_w
    assert n_sg % _NSET == 0 and n_sg > _NSET

    mesh = plsc.VectorSubcoreMesh(
        core_axis_name="c", subcore_axis_name="s", num_cores=_NC, num_subcores=_NS
    )

    @functools.partial(
        pl.kernel,
        out_type=jax.ShapeDtypeStruct((n, d), jnp.float32),
        mesh=mesh,
        compiler_params=pltpu.CompilerParams(use_tc_tiling_on_sc=False),
        scratch_types=[
            pltpu.VMEM((n_chunks, _C), jnp.int32),
            pltpu.VMEM((_NSET, _M, d), jnp.float32),
            [pltpu.SemaphoreType.DMA] * _NSET,
            [pltpu.SemaphoreType.DMA] * _NSET,
        ],
    )
    def gather_kernel(table_hbm, idx_hbm, out_hbm, idx_v, rows_v, gsems, wsems):
        wid = lax.axis_index("s") * _NC + lax.axis_index("c")
        base = wid * per_w
        pltpu.sync_copy(idx_hbm.at[wid], idx_v)

        def issue_gathers(sg, s):
            for k in range(_K):
                pltpu.async_copy(
                    table_hbm.at[idx_v.at[sg * _K + k]],
                    rows_v.at[s, pl.ds(k * _C, _C)],
                    gsems[s],
                )

        def wait_gathers(s):
            for k in range(_K):
                pltpu.make_async_copy(
                    table_hbm.at[idx_v.at[0]],
                    rows_v.at[s, pl.ds(k * _C, _C)],
                    gsems[s],
                ).wait()

        def issue_write(sg, s):
            pltpu.async_copy(
                rows_v.at[s], out_hbm.at[pl.ds(base + sg * _M, _M)], wsems[s]
            )

        def wait_write(s):
            pltpu.make_async_copy(
                rows_v.at[s], out_hbm.at[pl.ds(base, _M)], wsems[s]
            ).wait()

        for s in range(_LAG):
            issue_gathers(s, s)

        def group(j, carry):
            for u in range(_NSET):
                sg = j * _NSET + u
                wait_gathers(u)
                issue_write(sg, u)
                t = sg + _LAG
                s_t = (u + _LAG) % _NSET
                if u == 0:
                    @pl.when((t < n_sg) & (sg >= 1))
                    def _ww():
                        wait_write(s_t)
                else:
                    @pl.when(t < n_sg)
                    def _ww():
                        wait_write(s_t)

                @pl.when(t < n_sg)
                def _ig():
                    issue_gathers(t, s_t)

            return carry

        lax.fori_loop(0, n_sg // _NSET, group, 0)
        for u in range(_NSET):
            wait_write(u)

    return gather_kernel


def kernel(x, table):
    b, l = x.shape
    v, d = table.shape
    n = b * l
    per_w = n // _NW
    idx = x.reshape(_NW, per_w // _C, _C)
    out = _make_gather(n, v, d)(table, idx)
    return out.reshape(b, l, d)


# speedup vs baseline: 1.2267x; 1.2248x over previous
import functools

import jax
import jax.numpy as jnp
from jax import lax
from jax.experimental import pallas as pl
from jax.experimental.pallas import tpu as pltpu
from jax.experimental.pallas import tpu_sc as plsc

_NC = 2
_NS = 16
_NW = _NC * _NS

_W = 128
_C = 128
_NSET = 4
_LAG = _NSET - 1


@functools.lru_cache(maxsize=None)
def _make_gather(n, v):
    per_w = n // _NW
    n_sg = per_w // _C
    assert per_w * _NW == n and n_sg * _C == per_w
    assert n_sg % _NSET == 0 and n_sg > _NSET

    mesh = plsc.VectorSubcoreMesh(
        core_axis_name="c", subcore_axis_name="s", num_cores=_NC, num_subcores=_NS
    )

    @functools.partial(
        pl.kernel,
        out_type=jax.ShapeDtypeStruct((n, _W), jnp.float32),
        mesh=mesh,
        compiler_params=pltpu.CompilerParams(use_tc_tiling_on_sc=False),
        scratch_types=[
            pltpu.VMEM((n_sg, _C), jnp.int32),
            pltpu.VMEM((_NSET, _C, _W), jnp.float32),
            [pltpu.SemaphoreType.DMA] * _NSET,
            [pltpu.SemaphoreType.DMA] * _NSET,
        ],
    )
    def gather_kernel(table_hbm, idx_hbm, out_hbm, idx_v, rows_v, gsems, wsems):
        wid = lax.axis_index("s") * _NC + lax.axis_index("c")
        base = wid * per_w
        pltpu.sync_copy(idx_hbm.at[wid], idx_v)

        def issue_gather(g, s):
            pltpu.async_copy(table_hbm.at[idx_v.at[g]], rows_v.at[s], gsems[s])

        def wait_gather(s):
            pltpu.make_async_copy(
                table_hbm.at[idx_v.at[0]], rows_v.at[s], gsems[s]
            ).wait()

        def issue_write(g, s):
            pltpu.async_copy(
                rows_v.at[s], out_hbm.at[pl.ds(base + g * _C, _C)], wsems[s]
            )

        def wait_write(s):
            pltpu.make_async_copy(
                rows_v.at[s], out_hbm.at[pl.ds(base, _C)], wsems[s]
            ).wait()

        for s in range(_LAG):
            issue_gather(s, s)

        def group(j, carry):
            for u in range(_NSET):
                sg = j * _NSET + u
                wait_gather(u)
                issue_write(sg, u)
                t = sg + _LAG
                s_t = (u + _LAG) % _NSET
                if u == 0:
                    @pl.when((t < n_sg) & (sg >= 1))
                    def _ww():
                        wait_write(s_t)
                else:
                    @pl.when(t < n_sg)
                    def _ww():
                        wait_write(s_t)

                @pl.when(t < n_sg)
                def _ig():
                    issue_gather(t, s_t)

            return carry

        lax.fori_loop(0, n_sg // _NSET, group, 0)
        for u in range(_NSET):
            wait_write(u)

    return gather_kernel


def kernel(x, table):
    b, l = x.shape
    v, d = table.shape
    n = b * l
    per_w = n // _NW
    table_p = jnp.pad(table, ((0, 0), (0, _W - d)))
    idx = x.reshape(_NW, per_w // _C, _C)
    out_p = _make_gather(n, v)(table_p, idx)
    return out_p[:, :d].reshape(b, l, d)


# speedup vs baseline: 1.3232x; 1.0787x over previous
import functools

import jax
import jax.numpy as jnp
from jax import lax
from jax.experimental import pallas as pl
from jax.experimental.pallas import tpu as pltpu
from jax.experimental.pallas import tpu_sc as plsc

_NC = 2
_NS = 16
_NW = _NC * _NS

_W = 128
_C = 128
_NSET = 4
_LAG = _NSET - 1


@functools.lru_cache(maxsize=None)
def _make_gather(n, v, d):
    per_w = n // _NW
    n_sg = per_w // _C
    assert per_w * _NW == n and n_sg * _C == per_w
    assert n_sg % _NSET == 0 and n_sg > _NSET

    mesh = plsc.VectorSubcoreMesh(
        core_axis_name="c", subcore_axis_name="s", num_cores=_NC, num_subcores=_NS
    )

    @functools.partial(
        pl.kernel,
        out_type=jax.ShapeDtypeStruct((n, _W), jnp.float32),
        mesh=mesh,
        compiler_params=pltpu.CompilerParams(use_tc_tiling_on_sc=False),
        scratch_types=[
            pltpu.VMEM((n_sg, _C), jnp.int32),
            pltpu.VMEM((_NSET, _C, _W), jnp.float32),
            [pltpu.SemaphoreType.DMA] * _NSET,
            [pltpu.SemaphoreType.DMA] * _NSET,
        ],
    )
    def gather_kernel(table_hbm, idx_hbm, out_hbm, idx_v, rows_v, gsems, wsems):
        wid = lax.axis_index("s") * _NC + lax.axis_index("c")
        base = wid * per_w
        pltpu.sync_copy(idx_hbm.at[wid], idx_v)

        def issue_gather(g, s):
            pltpu.async_copy(table_hbm.at[idx_v.at[g]], rows_v.at[s], gsems[s])

        def wait_gather(s):
            pltpu.make_async_copy(
                table_hbm.at[idx_v.at[0]], rows_v.at[s], gsems[s]
            ).wait()

        def issue_write(g, s):
            pltpu.async_copy(
                rows_v.at[s, pl.ds(0, _C), pl.ds(0, d)],
                out_hbm.at[pl.ds(base + g * _C, _C), pl.ds(0, d)],
                wsems[s],
            )

        def wait_write(s):
            pltpu.make_async_copy(
                rows_v.at[s, pl.ds(0, _C), pl.ds(0, d)],
                out_hbm.at[pl.ds(base, _C), pl.ds(0, d)],
                wsems[s],
            ).wait()

        for s in range(_LAG):
            issue_gather(s, s)

        def group(j, carry):
            for u in range(_NSET):
                sg = j * _NSET + u
                wait_gather(u)
                issue_write(sg, u)
                t = sg + _LAG
                s_t = (u + _LAG) % _NSET
                if u == 0:
                    @pl.when((t < n_sg) & (sg >= 1))
                    def _ww():
                        wait_write(s_t)
                else:
                    @pl.when(t < n_sg)
                    def _ww():
                        wait_write(s_t)

                @pl.when(t < n_sg)
                def _ig():
                    issue_gather(t, s_t)

            return carry

        lax.fori_loop(0, n_sg // _NSET, group, 0)
        for u in range(_NSET):
            wait_write(u)

    return gather_kernel


def kernel(x, table):
    b, l = x.shape
    v, d = table.shape
    n = b * l
    per_w = n // _NW
    table_p = jnp.pad(table, ((0, 0), (0, _W - d)))
    idx = x.reshape(_NW, per_w // _C, _C)
    out_p = _make_gather(n, v, d)(table_p, idx)
    return out_p[:, :d].reshape(b, l, d)
